# baseline (speedup 1.0000x reference)
import functools

import jax
import jax.numpy as jnp
from jax import lax
from jax.experimental import pallas as pl
from jax.experimental.pallas import tpu as pltpu
from jax.experimental.pallas import tpu_sc as plsc

_NUM_WORKERS = 32


@functools.partial(jax.jit, static_argnames=())
def kernel(input, table):
    flat = input.reshape(-1).astype(jnp.int32)
    b_total = flat.shape[0]
    d = table.shape[1]
    bpw = b_total // _NUM_WORKERS
    ch = 128
    n_chunks = bpw // ch
    nbuf = min(8, n_chunks)

    mesh = plsc.VectorSubcoreMesh(core_axis_name="c", subcore_axis_name="s")

    @functools.partial(
        pl.kernel,
        mesh=mesh,
        compiler_params=pltpu.CompilerParams(use_tc_tiling_on_sc=False),
        out_type=jax.ShapeDtypeStruct((b_total // 128, 128, 2 * d), jnp.float32),
        scratch_types=(
            [pltpu.VMEM((bpw,), jnp.int32)]
            + [pltpu.VMEM((ch, d), jnp.float32) for _ in range(nbuf)]
            + [pltpu.SemaphoreType.DMA for _ in range(2 * nbuf)]
        ),
    )
    def gather_kernel(idx_hbm, table_hbm, out_hbm, idx_v, *bufs_and_sems):
        rows = bufs_and_sems[:nbuf]
        gsem = bufs_and_sems[nbuf:2 * nbuf]
        wsem = bufs_and_sems[2 * nbuf:3 * nbuf]

        wid = lax.axis_index("s") * 2 + lax.axis_index("c")
        base = wid * bpw
        pltpu.sync_copy(idx_hbm.at[pl.ds(base, bpw)], idx_v)

        def gather(c, b):
            return pltpu.async_copy(
                table_hbm.at[idx_v.at[pl.ds(c * ch, ch)]], rows[b], gsem[b])

        def write(c, b):
            return pltpu.async_copy(
                rows[b],
                out_hbm.at[(base + c * ch) // 128, :, pl.ds(0, d)], wsem[b])

        g = [gather(k, k) for k in range(nbuf)]
        w = [None] * nbuf
        for c in range(n_chunks):
            b = c % nbuf
            g[b].wait()
            w[b] = write(c, b)
            nc = c + nbuf
            if nc < n_chunks:
                w[b].wait()
                g[b] = gather(nc, b)
        for k in range(max(0, n_chunks - nbuf), n_chunks):
            w[k % nbuf].wait()

    tbl2 = jnp.pad(table, ((0, 0), (0, d))).reshape(2 * table.shape[0], d)
    outp = gather_kernel(flat * 2, tbl2)
    return outp[:, :, 0:d].reshape(b_total, d)
